# Initial kernel scaffold; baseline (speedup 1.0000x reference)
import jax
import jax.numpy as jnp
from jax.experimental import pallas as pl


def kernel(data, segment_ids):
    raise NotImplementedError("write your pallas kernel here")


# speedup vs baseline: 3.5348x; 3.5348x over previous
import functools

import jax
import jax.numpy as jnp
from jax import lax
from jax.experimental import pallas as pl
from jax.experimental.pallas import tpu as pltpu
from jax.experimental.pallas import tpu_sc as plsc

NUM_SEGMENTS = 10000
NCORE = 2
NSUB = 16
SHALF = NUM_SEGMENTS // NCORE

IDS_MINOR = 80
CHUNK = 400
SPAD = 5120
WB = 160
DUMMY = SHALF


def _sc_partials(data, ids1d):
    n, d = data.shape
    rows_per = n // NSUB
    nchunk = rows_per // CHUNK
    sub = CHUNK // IDS_MINOR
    s_rows = SPAD // NSUB

    mesh = plsc.VectorSubcoreMesh(core_axis_name="c", subcore_axis_name="s")

    @functools.partial(
        pl.kernel,
        out_type=jax.ShapeDtypeStruct((NCORE, SPAD, d), jnp.float32),
        mesh=mesh,
        scratch_types=[
            pltpu.VMEM((CHUNK, d), jnp.float32),
            pltpu.VMEM((CHUNK,), jnp.int32),
            pltpu.VMEM((IDS_MINOR,), jnp.int32),
            pltpu.VMEM((WB, d), jnp.float32),
            pltpu.VMEM_SHARED((SPAD, d), jnp.float32),
        ],
    )
    def k(data_hbm, ids_hbm, out_hbm, rows_v, ids_v, idx_v, buf_v, acc_sh):
        c = lax.axis_index("c")
        s = lax.axis_index("s")
        lo = c * SHALF

        zeros16 = jnp.zeros((16,), jnp.float32)

        def zrow(r, carry):
            for j in range(d // 16):
                buf_v[r, pl.ds(j * 16, 16)] = zeros16
            return carry

        lax.fori_loop(0, WB, zrow, 0)
        for j in range(s_rows // WB):
            pltpu.sync_copy(buf_v, acc_sh.at[pl.ds(s * s_rows + j * WB, WB)])
        plsc.subcore_barrier()

        base = s * rows_per

        def chunk(kk, carry):
            off = base + kk * CHUNK
            pltpu.sync_copy(ids_hbm.at[pl.ds(off, CHUNK)], ids_v)
            cmin = ids_v[pl.ds(0, 16)][0]
            cmax = ids_v[pl.ds(CHUNK - 16, 16)][15]
            overlap = jnp.logical_and(cmax >= lo, cmin < lo + SHALF)

            @pl.when(overlap)
            def _():
                pltpu.sync_copy(data_hbm.at[pl.ds(off, CHUNK)], rows_v)
                for j in range(sub):
                    for g in range(IDS_MINOR // 16):
                        ids16 = ids_v[pl.ds(j * IDS_MINOR + g * 16, 16)]
                        rel = ids16 - lo
                        ok = jnp.logical_and(rel >= 0, rel < SHALF)
                        idx_v[pl.ds(g * 16, 16)] = jnp.where(ok, rel, DUMMY)
                    pltpu.sync_copy(
                        rows_v.at[pl.ds(j * IDS_MINOR, IDS_MINOR)],
                        acc_sh.at[idx_v],
                        add=True,
                    )

            return carry

        lax.fori_loop(0, nchunk, chunk, 0)
        plsc.subcore_barrier()

        for j in range(s_rows // WB):
            r0 = s * s_rows + j * WB
            pltpu.sync_copy(acc_sh.at[pl.ds(r0, WB)], buf_v)
            pltpu.sync_copy(buf_v, out_hbm.at[c, pl.ds(r0, WB)])

    return k(data, ids1d)


def _combine(partials):
    def body(p_ref, o_ref):
        o_ref[:SHALF] = p_ref[0, :SHALF]
        o_ref[SHALF:] = p_ref[1, :SHALF]

    d = partials.shape[-1]
    return pl.pallas_call(
        body,
        out_shape=jax.ShapeDtypeStruct((NUM_SEGMENTS, d), jnp.float32),
    )(partials)


def kernel(data, segment_ids):
    partials = _sc_partials(data, segment_ids)
    return _combine(partials)


# speedup vs baseline: 3.9285x; 1.1114x over previous
import functools

import jax
import jax.numpy as jnp
from jax import lax
from jax.experimental import pallas as pl
from jax.experimental.pallas import tpu as pltpu
from jax.experimental.pallas import tpu_sc as plsc

NUM_SEGMENTS = 10000
NCORE = 2
NSUB = 16
SHALF = NUM_SEGMENTS // NCORE

IDS_MINOR = 80
CHUNK = 160
NBUF = 2
SPAD = 5120
WB = 80
DUMMY = SHALF


def _sc_partials(data, ids1d):
    n, d = data.shape
    rows_per = n // NSUB
    nch = rows_per // CHUNK
    sub = CHUNK // IDS_MINOR
    ngrp = (nch + 15) // 16
    s_rows = SPAD // NSUB

    mesh = plsc.VectorSubcoreMesh(core_axis_name="c", subcore_axis_name="s")

    @functools.partial(
        pl.kernel,
        out_type=jax.ShapeDtypeStruct((NCORE, SPAD, d), jnp.float32),
        mesh=mesh,
        scratch_types=[
            [pltpu.VMEM((CHUNK, d), jnp.float32) for _ in range(NBUF)],
            [pltpu.VMEM((sub, IDS_MINOR), jnp.int32) for _ in range(NBUF)],
            pltpu.VMEM((rows_per,), jnp.int32),
            pltpu.VMEM((WB, d), jnp.float32),
            pltpu.VMEM_SHARED((SPAD, d), jnp.float32),
            [pltpu.SemaphoreType.DMA for _ in range(NBUF)],
            [pltpu.SemaphoreType.DMA for _ in range(NBUF)],
        ],
    )
    def k(data_hbm, ids_hbm, out_hbm, rows_v, idx_v, ids_v, buf_v, acc_sh,
          sem_d, sem_s):
        c = lax.axis_index("c")
        s = lax.axis_index("s")
        lo = c * SHALF

        zeros16 = jnp.zeros((16,), jnp.float32)

        def zrow(r, carry):
            for j in range(d // 16):
                buf_v[r, pl.ds(j * 16, 16)] = zeros16
            return carry

        lax.fori_loop(0, WB, zrow, 0)
        for j in range(s_rows // WB):
            pltpu.sync_copy(buf_v, acc_sh.at[pl.ds(s * s_rows + j * WB, WB)])
        plsc.subcore_barrier()

        base = s * rows_per
        pltpu.sync_copy(ids_hbm.at[pl.ds(base, rows_per)], ids_v)

        def scan_chunk(kk, carry):
            nb, na = carry
            first = ids_v[pl.ds(kk * CHUNK, 16)][0]
            last = ids_v[pl.ds(kk * CHUNK + CHUNK - 16, 16)][15]
            nb = nb + jnp.where(last < lo, 1, 0)
            na = na + jnp.where(first >= lo + SHALF, 1, 0)
            return (nb, na)

        n_below, n_above = lax.fori_loop(
            0, nch, scan_chunk, (jnp.int32(0), jnp.int32(0)))
        k_lo = n_below
        k_hi = nch - n_above
        n4 = (k_hi - k_lo + (NBUF - 1)) // NBUF
        ks_raw = k_hi - NBUF * n4
        k_start = jnp.where(ks_raw > 0, ks_raw, 0)

        def start_dma(kk, b):
            kk_eff = jnp.where(kk < nch, kk, nch - 1)
            off = base + kk_eff * CHUNK
            return pltpu.async_copy(
                data_hbm.at[pl.ds(off, CHUNK)], rows_v[b], sem_d[b])

        def wait_dma(b):
            pltpu.make_async_copy(
                data_hbm.at[pl.ds(0, CHUNK)], rows_v[b], sem_d[b]).wait()

        def drain_scatters(b):
            pltpu.make_async_copy(
                data_hbm.at[pl.ds(0, CHUNK)], rows_v[b], sem_s[b]).wait()

        @pl.when(n4 > 0)
        def _():
            for b in range(NBUF):
                start_dma(k_start + b, b)

        def body(i, carry):
            kbase = k_start + i * NBUF
            for b in range(NBUF):
                kk = kbase + b
                kk_eff = jnp.where(kk < nch, kk, nch - 1)
                oob_off = jnp.where(kk < nch, 0, 1 << 20)
                wait_dma(b)
                for j in range(sub):
                    for g in range(IDS_MINOR // 16):
                        ids16 = ids_v[
                            pl.ds(kk_eff * CHUNK + j * IDS_MINOR + g * 16, 16)]
                        rel = (ids16 - lo) + oob_off
                        ok = jnp.logical_and(rel >= 0, rel < SHALF)
                        idx_v[b][j, pl.ds(g * 16, 16)] = jnp.where(
                            ok, rel, DUMMY)
                for j in range(sub):
                    pltpu.async_copy(
                        rows_v[b].at[pl.ds(j * IDS_MINOR, IDS_MINOR)],
                        acc_sh.at[idx_v[b].at[j]],
                        sem_s[b],
                        add=True,
                    )
            for b in range(NBUF):
                drain_scatters(b)

                @pl.when(i < n4 - 1)
                def _():
                    start_dma(kbase + NBUF + b, b)

            return carry

        lax.fori_loop(0, n4, body, 0)
        plsc.subcore_barrier()

        for j in range(s_rows // WB):
            r0 = s * s_rows + j * WB
            pltpu.sync_copy(acc_sh.at[pl.ds(r0, WB)], buf_v)
            pltpu.sync_copy(buf_v, out_hbm.at[c, pl.ds(r0, WB)])

    return k(data, ids1d)


def _combine(partials):
    def body(p_ref, o_ref):
        o_ref[:SHALF] = p_ref[0, :SHALF]
        o_ref[SHALF:] = p_ref[1, :SHALF]

    d = partials.shape[-1]
    return pl.pallas_call(
        body,
        out_shape=jax.ShapeDtypeStruct((NUM_SEGMENTS, d), jnp.float32),
    )(partials)


def kernel(data, segment_ids):
    partials = _sc_partials(data, segment_ids)
    return _combine(partials)


# speedup vs baseline: 5.4630x; 1.3906x over previous
import functools

import jax
import jax.numpy as jnp
from jax import lax
from jax.experimental import pallas as pl
from jax.experimental.pallas import tpu as pltpu
from jax.experimental.pallas import tpu_sc as plsc

NUM_SEGMENTS = 10000
NCORE = 2
NSUB = 16
SHALF = NUM_SEGMENTS // NCORE

IDS_MINOR = 80
CHUNK = 80
NBUF = 4
SPAD = 5120
WB = 80
DUMMY = SHALF


def _sc_partials(data, ids1d):
    n, d = data.shape
    rows_per = n // NSUB
    nch = rows_per // CHUNK
    sub = CHUNK // IDS_MINOR
    ngrp = (nch + 15) // 16
    s_rows = SPAD // NSUB

    mesh = plsc.VectorSubcoreMesh(core_axis_name="c", subcore_axis_name="s")

    @functools.partial(
        pl.kernel,
        out_type=jax.ShapeDtypeStruct((NCORE, SPAD, d), jnp.float32),
        mesh=mesh,
        scratch_types=[
            [pltpu.VMEM((CHUNK, d), jnp.float32) for _ in range(NBUF)],
            [pltpu.VMEM((sub, IDS_MINOR), jnp.int32) for _ in range(NBUF)],
            pltpu.VMEM((rows_per,), jnp.int32),
            pltpu.VMEM((WB, d), jnp.float32),
            pltpu.VMEM_SHARED((SPAD, d), jnp.float32),
            [pltpu.SemaphoreType.DMA for _ in range(NBUF)],
            [pltpu.SemaphoreType.DMA for _ in range(NBUF)],
            pltpu.SemaphoreType.DMA,
        ],
    )
    def k(data_hbm, ids_hbm, out_hbm, rows_v, idx_v, ids_v, buf_v, acc_sh,
          sem_d, sem_s, sem_i):
        c = lax.axis_index("c")
        s = lax.axis_index("s")
        lo = c * SHALF
        base = s * rows_per

        ids_copy = pltpu.async_copy(
            ids_hbm.at[pl.ds(base, rows_per)], ids_v, sem_i)

        zeros16 = jnp.zeros((16,), jnp.float32)

        def zrow(r, carry):
            for j in range(d // 16):
                buf_v[r, pl.ds(j * 16, 16)] = zeros16
            return carry

        lax.fori_loop(0, WB, zrow, 0)
        for j in range(s_rows // WB):
            pltpu.sync_copy(buf_v, acc_sh.at[pl.ds(s * s_rows + j * WB, WB)])
        plsc.subcore_barrier()
        ids_copy.wait()

        def scan_chunk(kk, carry):
            nb, na = carry
            first = ids_v[pl.ds(kk * CHUNK, 16)][0]
            last = ids_v[pl.ds(kk * CHUNK + CHUNK - 16, 16)][15]
            nb = nb + jnp.where(last < lo, 1, 0)
            na = na + jnp.where(first >= lo + SHALF, 1, 0)
            return (nb, na)

        n_below, n_above = lax.fori_loop(
            0, nch, scan_chunk, (jnp.int32(0), jnp.int32(0)))
        k_lo = n_below
        k_hi = nch - n_above
        n4 = (k_hi - k_lo + (NBUF - 1)) // NBUF
        ks_raw = k_hi - NBUF * n4
        k_start = jnp.where(ks_raw > 0, ks_raw, 0)

        def start_dma(kk, b):
            kk_eff = jnp.where(kk < nch, kk, nch - 1)
            off = base + kk_eff * CHUNK
            return pltpu.async_copy(
                data_hbm.at[pl.ds(off, CHUNK)], rows_v[b], sem_d[b])

        def wait_dma(b):
            pltpu.make_async_copy(
                data_hbm.at[pl.ds(0, CHUNK)], rows_v[b], sem_d[b]).wait()

        def drain_scatters(b):
            pltpu.make_async_copy(
                data_hbm.at[pl.ds(0, CHUNK)], rows_v[b], sem_s[b]).wait()

        @pl.when(n4 > 0)
        def _():
            for b in range(NBUF):
                start_dma(k_start + b, b)

        def body(i, carry):
            kbase = k_start + i * NBUF
            for b in range(NBUF):
                kk = kbase + b
                kk_eff = jnp.where(kk < nch, kk, nch - 1)
                oob_off = jnp.where(kk < nch, 0, 1 << 20)
                wait_dma(b)
                for j in range(sub):
                    for g in range(IDS_MINOR // 16):
                        ids16 = ids_v[
                            pl.ds(kk_eff * CHUNK + j * IDS_MINOR + g * 16, 16)]
                        rel = (ids16 - lo) + oob_off
                        ok = jnp.logical_and(rel >= 0, rel < SHALF)
                        idx_v[b][j, pl.ds(g * 16, 16)] = jnp.where(
                            ok, rel, DUMMY)
                for j in range(sub):
                    pltpu.async_copy(
                        rows_v[b].at[pl.ds(j * IDS_MINOR, IDS_MINOR)],
                        acc_sh.at[idx_v[b].at[j]],
                        sem_s[b],
                        add=True,
                    )
            for b in range(NBUF):
                drain_scatters(b)

                @pl.when(i < n4 - 1)
                def _():
                    start_dma(kbase + NBUF + b, b)

            return carry

        lax.fori_loop(0, n4, body, 0)
        plsc.subcore_barrier()

        for j in range(s_rows // WB):
            r0 = s * s_rows + j * WB
            pltpu.sync_copy(acc_sh.at[pl.ds(r0, WB)], buf_v)
            pltpu.sync_copy(buf_v, out_hbm.at[c, pl.ds(r0, WB)])

    return k(data, ids1d)


def _combine(partials):
    def body(p_ref, o_ref):
        o_ref[:SHALF] = p_ref[0, :SHALF]
        o_ref[SHALF:] = p_ref[1, :SHALF]

    d = partials.shape[-1]
    return pl.pallas_call(
        body,
        out_shape=jax.ShapeDtypeStruct((NUM_SEGMENTS, d), jnp.float32),
    )(partials)


def kernel(data, segment_ids):
    partials = _sc_partials(data, segment_ids)
    return _combine(partials)
